# baseline (speedup 1.0000x reference)
import jax
import jax.numpy as jnp
from jax.experimental import pallas as pl
from jax.experimental.pallas import tpu as pltpu

_USER_CHUNKS = 8
_ITEM_CHUNKS = 2


def _copy_body(user_in, item_in, user_out, item_out, *sems):
    copies = []
    n_user = user_in.shape[0]
    rows = n_user // _USER_CHUNKS
    for c in range(_USER_CHUNKS):
        copies.append(pltpu.make_async_copy(
            user_in.at[pl.ds(c * rows, rows)],
            user_out.at[pl.ds(c * rows, rows)],
            sems[c]))
    n_item = item_in.shape[0]
    rows = n_item // _ITEM_CHUNKS
    for c in range(_ITEM_CHUNKS):
        copies.append(pltpu.make_async_copy(
            item_in.at[pl.ds(c * rows, rows)],
            item_out.at[pl.ds(c * rows, rows)],
            sems[_USER_CHUNKS + c]))
    for cp in copies:
        cp.start()
    for cp in copies:
        cp.wait()


def kernel(emb_user, emb_item):
    return pl.pallas_call(
        _copy_body,
        out_shape=(
            jax.ShapeDtypeStruct(emb_user.shape, emb_user.dtype),
            jax.ShapeDtypeStruct(emb_item.shape, emb_item.dtype),
        ),
        in_specs=[
            pl.BlockSpec(memory_space=pl.ANY),
            pl.BlockSpec(memory_space=pl.ANY),
        ],
        out_specs=(
            pl.BlockSpec(memory_space=pl.ANY),
            pl.BlockSpec(memory_space=pl.ANY),
        ),
        scratch_shapes=[pltpu.SemaphoreType.DMA] * (_USER_CHUNKS + _ITEM_CHUNKS),
    )(emb_user, emb_item)


# speedup vs baseline: 1.8147x; 1.8147x over previous
import jax
import jax.numpy as jnp
from jax.experimental import pallas as pl
from jax.experimental.pallas import tpu as pltpu

_USER_CHUNKS = 10
_ITEM_CHUNKS = 2


def _copy_body(user_in, item_in, user_out, item_out, *sems):
    copies = []
    n_user = user_in.shape[0]
    rows = n_user // _USER_CHUNKS
    for c in range(_USER_CHUNKS):
        copies.append(pltpu.make_async_copy(
            user_in.at[pl.ds(c * rows, rows)],
            user_out.at[pl.ds(c * rows, rows)],
            sems[c]))
    n_item = item_in.shape[0]
    rows = n_item // _ITEM_CHUNKS
    for c in range(_ITEM_CHUNKS):
        copies.append(pltpu.make_async_copy(
            item_in.at[pl.ds(c * rows, rows)],
            item_out.at[pl.ds(c * rows, rows)],
            sems[_USER_CHUNKS + c]))
    for cp in copies:
        cp.start()
    for cp in copies:
        cp.wait()


def kernel(emb_user, emb_item):
    u_shape, i_shape = emb_user.shape, emb_item.shape
    u2 = emb_user.reshape(-1, 128)
    i2 = emb_item.reshape(-1, 128)
    out_u, out_i = pl.pallas_call(
        _copy_body,
        out_shape=(
            jax.ShapeDtypeStruct(u2.shape, u2.dtype),
            jax.ShapeDtypeStruct(i2.shape, i2.dtype),
        ),
        in_specs=[
            pl.BlockSpec(memory_space=pl.ANY),
            pl.BlockSpec(memory_space=pl.ANY),
        ],
        out_specs=(
            pl.BlockSpec(memory_space=pl.ANY),
            pl.BlockSpec(memory_space=pl.ANY),
        ),
        scratch_shapes=[pltpu.SemaphoreType.DMA] * (_USER_CHUNKS + _ITEM_CHUNKS),
    )(u2, i2)
    return (out_u.reshape(u_shape), out_i.reshape(i_shape))


# speedup vs baseline: 12.3087x; 6.7827x over previous
import jax
import jax.numpy as jnp
from jax.experimental import pallas as pl
from jax.experimental.pallas import tpu as pltpu


def _copy_body(in_ref, out_ref):
    out_ref[...] = in_ref[...]


def _copy(x, block_rows):
    n_rows, width = x.shape
    grid = n_rows // block_rows
    return pl.pallas_call(
        _copy_body,
        out_shape=jax.ShapeDtypeStruct(x.shape, x.dtype),
        grid=(grid,),
        in_specs=[pl.BlockSpec((block_rows, width), lambda i: (i, 0))],
        out_specs=pl.BlockSpec((block_rows, width), lambda i: (i, 0)),
    )(x)


def kernel(emb_user, emb_item):
    u_shape, i_shape = emb_user.shape, emb_item.shape
    u2 = emb_user.reshape(-1, 128)
    i2 = emb_item.reshape(-1, 128)
    out_u = _copy(u2, 10000)
    out_i = _copy(i2, 10000)
    return (out_u.reshape(u_shape), out_i.reshape(i_shape))


# speedup vs baseline: 14.5561x; 1.1826x over previous
import jax
import jax.numpy as jnp
from jax.experimental import pallas as pl
from jax.experimental.pallas import tpu as pltpu

_NBUF = 8
_CHUNK = 10000


def _copy_body(u_in, i_in, u_out, i_out, bufs, in_sems, out_sems):
    n_u = u_in.shape[0] // _CHUNK
    n_i = i_in.shape[0] // _CHUNK
    chunks = [(u_in, u_out, r) for r in range(n_u)]
    chunks += [(i_in, i_out, r) for r in range(n_i)]
    n = len(chunks)

    def in_copy(c):
        src, _, r = chunks[c]
        slot = c % _NBUF
        return pltpu.make_async_copy(
            src.at[pl.ds(r * _CHUNK, _CHUNK)], bufs.at[slot], in_sems.at[slot])

    def out_copy(c):
        _, dst, r = chunks[c]
        slot = c % _NBUF
        return pltpu.make_async_copy(
            bufs.at[slot], dst.at[pl.ds(r * _CHUNK, _CHUNK)], out_sems.at[slot])

    for c in range(min(_NBUF, n)):
        in_copy(c).start()
    for c in range(n):
        in_copy(c).wait()
        out_copy(c).start()
        d = c - (_NBUF - 1)
        if d >= 0:
            out_copy(d).wait()
            if d + _NBUF < n:
                in_copy(d + _NBUF).start()
    for c in range(max(0, n - _NBUF + 1), n):
        out_copy(c).wait()


def kernel(emb_user, emb_item):
    width = emb_user.shape[1]
    return pl.pallas_call(
        _copy_body,
        out_shape=(
            jax.ShapeDtypeStruct(emb_user.shape, emb_user.dtype),
            jax.ShapeDtypeStruct(emb_item.shape, emb_item.dtype),
        ),
        in_specs=[
            pl.BlockSpec(memory_space=pl.ANY),
            pl.BlockSpec(memory_space=pl.ANY),
        ],
        out_specs=(
            pl.BlockSpec(memory_space=pl.ANY),
            pl.BlockSpec(memory_space=pl.ANY),
        ),
        scratch_shapes=[
            pltpu.VMEM((_NBUF, _CHUNK, width), jnp.float32),
            pltpu.SemaphoreType.DMA((_NBUF,)),
            pltpu.SemaphoreType.DMA((_NBUF,)),
        ],
    )(emb_user, emb_item)


# speedup vs baseline: 16.1856x; 1.1119x over previous
import jax
import jax.numpy as jnp
from jax.experimental import pallas as pl
from jax.experimental.pallas import tpu as pltpu

_NBUF = 8
_LAG = 4
_CHUNK = 10000


def _copy_body(u_in, i_in, u_out, i_out, bufs, in_sems, out_sems):
    n_u = u_in.shape[0] // _CHUNK
    n_i = i_in.shape[0] // _CHUNK
    chunks = [(u_in, u_out, r) for r in range(n_u)]
    chunks += [(i_in, i_out, r) for r in range(n_i)]
    n = len(chunks)

    def in_copy(c):
        src, _, r = chunks[c]
        slot = c % _NBUF
        return pltpu.make_async_copy(
            src.at[pl.ds(r * _CHUNK, _CHUNK)], bufs.at[slot], in_sems.at[slot])

    def out_copy(c):
        _, dst, r = chunks[c]
        slot = c % _NBUF
        return pltpu.make_async_copy(
            bufs.at[slot], dst.at[pl.ds(r * _CHUNK, _CHUNK)], out_sems.at[slot])

    for c in range(min(_NBUF, n)):
        in_copy(c).start()
    for c in range(n):
        in_copy(c).wait()
        out_copy(c).start()
        d = c - _LAG
        if d >= 0:
            out_copy(d).wait()
            if d + _NBUF < n:
                in_copy(d + _NBUF).start()
    for c in range(max(0, n - _LAG), n):
        out_copy(c).wait()


def kernel(emb_user, emb_item):
    width = emb_user.shape[1]
    return pl.pallas_call(
        _copy_body,
        out_shape=(
            jax.ShapeDtypeStruct(emb_user.shape, emb_user.dtype),
            jax.ShapeDtypeStruct(emb_item.shape, emb_item.dtype),
        ),
        in_specs=[
            pl.BlockSpec(memory_space=pl.ANY),
            pl.BlockSpec(memory_space=pl.ANY),
        ],
        out_specs=(
            pl.BlockSpec(memory_space=pl.ANY),
            pl.BlockSpec(memory_space=pl.ANY),
        ),
        scratch_shapes=[
            pltpu.VMEM((_NBUF, _CHUNK, width), jnp.float32),
            pltpu.SemaphoreType.DMA((_NBUF,)),
            pltpu.SemaphoreType.DMA((_NBUF,)),
        ],
    )(emb_user, emb_item)
